# baseline (speedup 1.0000x reference)
import functools
import math

import jax
import jax.numpy as jnp
from jax import lax
from jax.experimental import pallas as pl
from jax.experimental.pallas import tpu as pltpu

_B, _N, _K = 4, 4096, 16


def _bn(x, p):
    return x * p["g"] + p["be"]


def _lrelu(x, s):
    return jnp.where(x >= 0.0, x, s * x)


def _apply_mlp(x, layers):
    for l in layers:
        x = _bn(jax.nn.relu(x @ l["W"] + l["b"]), l)
    return x


def _pdist2(a, b):
    return jnp.sum(a * a, -1)[:, None] + jnp.sum(b * b, -1)[None, :] - 2.0 * (a @ b.T)


def _knn(coords, k):
    d2 = _pdist2(coords, coords)
    neg, idx = jax.lax.top_k(-d2, k)
    d = jnp.maximum(-neg, 0.0)
    dist = jnp.where(d > 1e-12, jnp.sqrt(jnp.where(d > 1e-12, d, 1.0)), 0.0)
    return idx, dist


def _fps(pos, m):
    n = pos.shape[0]
    idxs = jnp.zeros((m,), jnp.int32)
    dists = jnp.full((n,), jnp.inf, dtype=jnp.float32)

    def body(i, st):
        ii, dd = st
        last = pos[ii[i - 1]]
        dd = jnp.minimum(dd, jnp.sum((pos - last) ** 2, -1))
        ii = ii.at[i].set(jnp.argmax(dd).astype(jnp.int32))
        return (ii, dd)

    idxs, _ = jax.lax.fori_loop(1, m, body, (idxs, dists))
    return idxs


def _sa_module(x, pos, ratio, r, layers):
    b, n, _ = x.shape
    m = max(1, int(n * ratio))
    idx = jax.vmap(lambda p: _fps(p, m))(jax.lax.stop_gradient(pos))
    qpos = jnp.take_along_axis(pos, idx[:, :, None], axis=1)
    d2 = jax.vmap(_pdist2)(qpos, pos)
    kk = min(64, n)
    negd, nidx = jax.lax.top_k(-d2, kk)
    valid = (-negd) <= r * r
    xj = jax.vmap(lambda a, i: a[i])(x, nidx)
    pj = jax.vmap(lambda a, i: a[i])(pos, nidx)
    msg = _apply_mlp(jnp.concatenate([xj, pj - qpos[:, :, None, :]], -1), layers)
    msg = jnp.where(valid[..., None], msg, -1e30)
    out = jnp.max(msg, axis=2)
    return out, qpos


def _lse(coords, feats, idx, dist, p):
    nbr = jax.vmap(lambda c, i: c[i])(coords, idx)
    ctr = jnp.broadcast_to(coords[:, :, None, :], nbr.shape)
    cat = jnp.concatenate([ctr, nbr, ctr - nbr, dist[..., None]], -1)
    enc = _bn(jax.nn.relu(cat @ p["W"] + p["b"]), p)
    f = jnp.broadcast_to(feats[:, :, None, :], enc.shape)
    return jnp.concatenate([enc, f], -1)


def _att_pool(x, Ws, mp):
    scores = jax.nn.softmax(x @ Ws, axis=2)
    pooled = jnp.sum(scores * x, axis=2)
    return _bn(jax.nn.relu(pooled @ mp["W"] + mp["b"]), mp)


def _copy_kernel(x_ref, o_ref):
    o_ref[...] = x_ref[...]


def _pl_identity(x):
    return pl.pallas_call(
        _copy_kernel,
        out_shape=jax.ShapeDtypeStruct(x.shape, x.dtype),
    )(x)


def kernel(data, params):
    p = params
    coords = data[..., :3]
    local = data[..., 3:]
    x = local @ p["fc_start"]["W"] + p["fc_start"]["b"]
    x = _lrelu(_bn(x, p["bn_start"]), 0.2)
    knn_idx, knn_dist = jax.vmap(lambda c: _knn(c, _K))(coords)
    features = x
    x = _lrelu(x @ p["mlp1"]["W"] + p["mlp1"]["b"], 0.2)
    x = _lse(coords, x, knn_idx, knn_dist, p["lse1"])
    x = _att_pool(x, p["pool1_score"], p["pool1_mlp"])
    x = _lse(coords, x, knn_idx, knn_dist, p["lse2"])
    x = _att_pool(x, p["pool2_score"], p["pool2_mlp"])
    x = _lrelu((x @ p["mlp2"]["W"] + p["mlp2"]["b"])
               + _bn(features @ p["shortcut"]["W"] + p["shortcut"]["b"], p["shortcut"]), 0.01)
    x = jnp.reshape(jnp.swapaxes(x, 1, 2), (x.shape[0], x.shape[1], 32))
    x, pos = _sa_module(x, coords, 0.2, 0.2, p["sa1"])
    x, pos = _sa_module(x, pos, 0.5, 0.2, p["sa1a"])
    x, pos = _sa_module(x, pos, 0.25, 0.4, p["sa2"])
    h = _apply_mlp(jnp.concatenate([x, pos], -1), p["sa3"])
    g = jnp.max(h, axis=1)
    g = jax.nn.relu(g @ p["lin1"]["W"] + p["lin1"]["b"])
    g = jax.nn.relu(g @ p["lin2"]["W"] + p["lin2"]["b"])
    g = jax.nn.relu(g @ p["lin3"]["W"] + p["lin3"]["b"])
    g = g @ p["lin4"]["W"] + p["lin4"]["b"]
    return _pl_identity(jax.nn.log_softmax(g, axis=-1))


# speedup vs baseline: 1.2837x; 1.2837x over previous
import functools
import math

import jax
import jax.numpy as jnp
from jax import lax
from jax.experimental import pallas as pl
from jax.experimental.pallas import tpu as pltpu

_B, _N, _K = 4, 4096, 16


def _bn(x, p):
    return x * p["g"] + p["be"]


def _lrelu(x, s):
    return jnp.where(x >= 0.0, x, s * x)


def _apply_mlp(x, layers):
    for l in layers:
        x = _bn(jax.nn.relu(x @ l["W"] + l["b"]), l)
    return x


def _pdist2(a, b):
    return jnp.sum(a * a, -1)[:, None] + jnp.sum(b * b, -1)[None, :] - 2.0 * (a @ b.T)


def _knn(coords, k):
    d2 = _pdist2(coords, coords)
    neg, idx = jax.lax.top_k(-d2, k)
    d = jnp.maximum(-neg, 0.0)
    dist = jnp.where(d > 1e-12, jnp.sqrt(jnp.where(d > 1e-12, d, 1.0)), 0.0)
    return idx, dist


def _fps_body(posT_ref, qpos_ref, dd_ref):
    B, _, n = posT_ref.shape
    m = qpos_ref.shape[1]
    px = posT_ref[:, 0, :]
    py = posT_ref[:, 1, :]
    pz = posT_ref[:, 2, :]
    lanes = lax.broadcasted_iota(jnp.int32, (B, n), 1)

    def extract(j):
        sel = lanes == j[:, None]
        lx = jnp.sum(jnp.where(sel, px, 0.0), axis=1)
        ly = jnp.sum(jnp.where(sel, py, 0.0), axis=1)
        lz = jnp.sum(jnp.where(sel, pz, 0.0), axis=1)
        return lx, ly, lz

    def store_q(i, lx, ly, lz):
        row = jnp.concatenate(
            [lx[:, None, None], ly[:, None, None], lz[:, None, None]], axis=2)
        qpos_ref[:, pl.ds(i, 1), :] = row

    dd_ref[...] = jnp.full((B, n), jnp.inf, jnp.float32)
    j0 = jnp.zeros((B,), jnp.int32)
    lx, ly, lz = extract(j0)
    store_q(0, lx, ly, lz)

    def body(i, carry):
        lx, ly, lz = carry
        d2 = ((px - lx[:, None]) ** 2 + (py - ly[:, None]) ** 2
              + (pz - lz[:, None]) ** 2)
        dd = jnp.minimum(dd_ref[...], d2)
        dd_ref[...] = dd
        mx = jnp.max(dd, axis=1)
        j = jnp.min(jnp.where(dd == mx[:, None], lanes, n), axis=1).astype(jnp.int32)
        lx, ly, lz = extract(j)
        store_q(i, lx, ly, lz)
        return (lx, ly, lz)

    lax.fori_loop(1, m, body, (lx, ly, lz))


def _fps_pallas(pos, m):
    B, n, _ = pos.shape
    posT = jnp.swapaxes(pos, 1, 2)
    return pl.pallas_call(
        _fps_body,
        out_shape=jax.ShapeDtypeStruct((B, m, 3), jnp.float32),
        scratch_shapes=[pltpu.VMEM((B, n), jnp.float32)],
    )(posT)


def _sa_module(x, pos, ratio, r, layers):
    b, n, _ = x.shape
    m = max(1, int(n * ratio))
    qpos = _fps_pallas(jax.lax.stop_gradient(pos), m)
    d2 = jax.vmap(_pdist2)(qpos, pos)
    kk = min(64, n)
    negd, nidx = jax.lax.top_k(-d2, kk)
    valid = (-negd) <= r * r
    xj = jax.vmap(lambda a, i: a[i])(x, nidx)
    pj = jax.vmap(lambda a, i: a[i])(pos, nidx)
    msg = _apply_mlp(jnp.concatenate([xj, pj - qpos[:, :, None, :]], -1), layers)
    msg = jnp.where(valid[..., None], msg, -1e30)
    out = jnp.max(msg, axis=2)
    return out, qpos


def _lse(coords, feats, idx, dist, p):
    nbr = jax.vmap(lambda c, i: c[i])(coords, idx)
    ctr = jnp.broadcast_to(coords[:, :, None, :], nbr.shape)
    cat = jnp.concatenate([ctr, nbr, ctr - nbr, dist[..., None]], -1)
    enc = _bn(jax.nn.relu(cat @ p["W"] + p["b"]), p)
    f = jnp.broadcast_to(feats[:, :, None, :], enc.shape)
    return jnp.concatenate([enc, f], -1)


def _att_pool(x, Ws, mp):
    scores = jax.nn.softmax(x @ Ws, axis=2)
    pooled = jnp.sum(scores * x, axis=2)
    return _bn(jax.nn.relu(pooled @ mp["W"] + mp["b"]), mp)


def _copy_kernel(x_ref, o_ref):
    o_ref[...] = x_ref[...]


def _pl_identity(x):
    return pl.pallas_call(
        _copy_kernel,
        out_shape=jax.ShapeDtypeStruct(x.shape, x.dtype),
    )(x)


def kernel(data, params):
    p = params
    coords = data[..., :3]
    local = data[..., 3:]
    x = local @ p["fc_start"]["W"] + p["fc_start"]["b"]
    x = _lrelu(_bn(x, p["bn_start"]), 0.2)
    knn_idx, knn_dist = jax.vmap(lambda c: _knn(c, _K))(coords)
    features = x
    x = _lrelu(x @ p["mlp1"]["W"] + p["mlp1"]["b"], 0.2)
    x = _lse(coords, x, knn_idx, knn_dist, p["lse1"])
    x = _att_pool(x, p["pool1_score"], p["pool1_mlp"])
    x = _lse(coords, x, knn_idx, knn_dist, p["lse2"])
    x = _att_pool(x, p["pool2_score"], p["pool2_mlp"])
    x = _lrelu((x @ p["mlp2"]["W"] + p["mlp2"]["b"])
               + _bn(features @ p["shortcut"]["W"] + p["shortcut"]["b"], p["shortcut"]), 0.01)
    x = jnp.reshape(jnp.swapaxes(x, 1, 2), (x.shape[0], x.shape[1], 32))
    x, pos = _sa_module(x, coords, 0.2, 0.2, p["sa1"])
    x, pos = _sa_module(x, pos, 0.5, 0.2, p["sa1a"])
    x, pos = _sa_module(x, pos, 0.25, 0.4, p["sa2"])
    h = _apply_mlp(jnp.concatenate([x, pos], -1), p["sa3"])
    g = jnp.max(h, axis=1)
    g = jax.nn.relu(g @ p["lin1"]["W"] + p["lin1"]["b"])
    g = jax.nn.relu(g @ p["lin2"]["W"] + p["lin2"]["b"])
    g = jax.nn.relu(g @ p["lin3"]["W"] + p["lin3"]["b"])
    g = g @ p["lin4"]["W"] + p["lin4"]["b"]
    return _pl_identity(jax.nn.log_softmax(g, axis=-1))


# speedup vs baseline: 2.5240x; 1.9662x over previous
import functools
import math

import jax
import jax.numpy as jnp
from jax import lax
from jax.experimental import pallas as pl
from jax.experimental.pallas import tpu as pltpu

_B, _N, _K = 4, 4096, 16


def _bn(x, p):
    return x * p["g"] + p["be"]


def _lrelu(x, s):
    return jnp.where(x >= 0.0, x, s * x)


def _apply_mlp(x, layers):
    for l in layers:
        x = _bn(jax.nn.relu(x @ l["W"] + l["b"]), l)
    return x


def _topk_body(pts_ref, qT_ref, qn_ref, idx_ref, val_ref, d2_ref, *, k, nb, flat_base):
    n = pts_ref.shape[1]
    R = qT_ref.shape[2]
    b = pl.program_id(0)
    pts = pts_ref[0]
    pn = jnp.sum(pts * pts, axis=1)
    mm = jnp.dot(pts, qT_ref[0], preferred_element_type=jnp.float32)
    d2_ref[...] = pn[:, None] + qn_ref[0] - 2.0 * mm
    subiota = lax.broadcasted_iota(jnp.int32, (n, R), 0)

    def step(s, _):
        d2c = d2_ref[...]
        mv = jnp.min(d2c, axis=0)
        li = jnp.min(jnp.where(d2c == mv[None, :], subiota, n), axis=0)
        d2_ref[...] = jnp.where(subiota == li[None, :], jnp.inf, d2c)
        val_ref[0, pl.ds(s, 1), :] = mv[None, :]
        gidx = li + b * nb if flat_base else li
        idx_ref[0, pl.ds(s, 1), :] = gidx[None, :]
        return 0

    lax.fori_loop(0, k, step, 0, unroll=False)


def _topk_pallas(q, pts, k, flat_base=True, R=128):
    B, m, _ = q.shape
    n = pts.shape[1]
    mp = ((m + R - 1) // R) * R
    qT = jnp.swapaxes(q, 1, 2)
    qn = jnp.sum(qT * qT, axis=1, keepdims=True)
    if mp != m:
        qT = jnp.pad(qT, ((0, 0), (0, 0), (0, mp - m)))
        qn = jnp.pad(qn, ((0, 0), (0, 0), (0, mp - m)))
    kernel = functools.partial(_topk_body, k=k, nb=n, flat_base=flat_base)
    idx, val = pl.pallas_call(
        kernel,
        grid=(B, mp // R),
        in_specs=[
            pl.BlockSpec((1, n, 3), lambda b, j: (b, 0, 0)),
            pl.BlockSpec((1, 3, R), lambda b, j: (b, 0, j)),
            pl.BlockSpec((1, 1, R), lambda b, j: (b, 0, j)),
        ],
        out_specs=[
            pl.BlockSpec((1, k, R), lambda b, j: (b, 0, j)),
            pl.BlockSpec((1, k, R), lambda b, j: (b, 0, j)),
        ],
        out_shape=[
            jax.ShapeDtypeStruct((B, k, mp), jnp.int32),
            jax.ShapeDtypeStruct((B, k, mp), jnp.float32),
        ],
        scratch_shapes=[pltpu.VMEM((n, R), jnp.float32)],
    )(pts, qT, qn)
    return idx[:, :, :m], val[:, :, :m]


def _knn(coords, k):
    idx, val = _topk_pallas(coords, coords, k, flat_base=False)
    idx = jnp.swapaxes(idx, 1, 2)
    d = jnp.maximum(jnp.swapaxes(val, 1, 2), 0.0)
    dist = jnp.where(d > 1e-12, jnp.sqrt(jnp.where(d > 1e-12, d, 1.0)), 0.0)
    return idx, dist


def _fps_body(posT_ref, qpos_ref, dd_ref):
    B, _, n = posT_ref.shape
    m = qpos_ref.shape[1]
    px = posT_ref[:, 0, :]
    py = posT_ref[:, 1, :]
    pz = posT_ref[:, 2, :]
    lanes = lax.broadcasted_iota(jnp.int32, (B, n), 1)

    def extract(j):
        sel = lanes == j[:, None]
        lx = jnp.sum(jnp.where(sel, px, 0.0), axis=1)
        ly = jnp.sum(jnp.where(sel, py, 0.0), axis=1)
        lz = jnp.sum(jnp.where(sel, pz, 0.0), axis=1)
        return lx, ly, lz

    def store_q(i, lx, ly, lz):
        row = jnp.concatenate(
            [lx[:, None, None], ly[:, None, None], lz[:, None, None]], axis=2)
        qpos_ref[:, pl.ds(i, 1), :] = row

    dd_ref[...] = jnp.full((B, n), jnp.inf, jnp.float32)
    j0 = jnp.zeros((B,), jnp.int32)
    lx, ly, lz = extract(j0)
    store_q(0, lx, ly, lz)

    def body(i, carry):
        lx, ly, lz = carry
        d2 = ((px - lx[:, None]) ** 2 + (py - ly[:, None]) ** 2
              + (pz - lz[:, None]) ** 2)
        dd = jnp.minimum(dd_ref[...], d2)
        dd_ref[...] = dd
        mx = jnp.max(dd, axis=1)
        j = jnp.min(jnp.where(dd == mx[:, None], lanes, n), axis=1).astype(jnp.int32)
        lx, ly, lz = extract(j)
        store_q(i, lx, ly, lz)
        return (lx, ly, lz)

    lax.fori_loop(1, m, body, (lx, ly, lz))


def _fps_pallas(pos, m):
    B, n, _ = pos.shape
    posT = jnp.swapaxes(pos, 1, 2)
    return pl.pallas_call(
        _fps_body,
        out_shape=jax.ShapeDtypeStruct((B, m, 3), jnp.float32),
        scratch_shapes=[pltpu.VMEM((B, n), jnp.float32)],
    )(posT)


def _sa_module(x, pos, ratio, r, layers):
    b, n, _ = x.shape
    m = max(1, int(n * ratio))
    qpos = _fps_pallas(jax.lax.stop_gradient(pos), m)
    kk = min(64, n)
    nidx, nval = _topk_pallas(qpos, pos, kk, flat_base=False)
    nidx = jnp.swapaxes(nidx, 1, 2)
    valid = jnp.swapaxes(nval, 1, 2) <= r * r
    xj = jax.vmap(lambda a, i: a[i])(x, nidx)
    pj = jax.vmap(lambda a, i: a[i])(pos, nidx)
    msg = _apply_mlp(jnp.concatenate([xj, pj - qpos[:, :, None, :]], -1), layers)
    msg = jnp.where(valid[..., None], msg, -1e30)
    out = jnp.max(msg, axis=2)
    return out, qpos


def _lse(coords, feats, idx, dist, p):
    nbr = jax.vmap(lambda c, i: c[i])(coords, idx)
    ctr = jnp.broadcast_to(coords[:, :, None, :], nbr.shape)
    cat = jnp.concatenate([ctr, nbr, ctr - nbr, dist[..., None]], -1)
    enc = _bn(jax.nn.relu(cat @ p["W"] + p["b"]), p)
    f = jnp.broadcast_to(feats[:, :, None, :], enc.shape)
    return jnp.concatenate([enc, f], -1)


def _att_pool(x, Ws, mp):
    scores = jax.nn.softmax(x @ Ws, axis=2)
    pooled = jnp.sum(scores * x, axis=2)
    return _bn(jax.nn.relu(pooled @ mp["W"] + mp["b"]), mp)


def _copy_kernel(x_ref, o_ref):
    o_ref[...] = x_ref[...]


def _pl_identity(x):
    return pl.pallas_call(
        _copy_kernel,
        out_shape=jax.ShapeDtypeStruct(x.shape, x.dtype),
    )(x)


def kernel(data, params):
    p = params
    coords = data[..., :3]
    local = data[..., 3:]
    x = local @ p["fc_start"]["W"] + p["fc_start"]["b"]
    x = _lrelu(_bn(x, p["bn_start"]), 0.2)
    knn_idx, knn_dist = _knn(coords, _K)
    features = x
    x = _lrelu(x @ p["mlp1"]["W"] + p["mlp1"]["b"], 0.2)
    x = _lse(coords, x, knn_idx, knn_dist, p["lse1"])
    x = _att_pool(x, p["pool1_score"], p["pool1_mlp"])
    x = _lse(coords, x, knn_idx, knn_dist, p["lse2"])
    x = _att_pool(x, p["pool2_score"], p["pool2_mlp"])
    x = _lrelu((x @ p["mlp2"]["W"] + p["mlp2"]["b"])
               + _bn(features @ p["shortcut"]["W"] + p["shortcut"]["b"], p["shortcut"]), 0.01)
    x = jnp.reshape(jnp.swapaxes(x, 1, 2), (x.shape[0], x.shape[1], 32))
    x, pos = _sa_module(x, coords, 0.2, 0.2, p["sa1"])
    x, pos = _sa_module(x, pos, 0.5, 0.2, p["sa1a"])
    x, pos = _sa_module(x, pos, 0.25, 0.4, p["sa2"])
    h = _apply_mlp(jnp.concatenate([x, pos], -1), p["sa3"])
    g = jnp.max(h, axis=1)
    g = jax.nn.relu(g @ p["lin1"]["W"] + p["lin1"]["b"])
    g = jax.nn.relu(g @ p["lin2"]["W"] + p["lin2"]["b"])
    g = jax.nn.relu(g @ p["lin3"]["W"] + p["lin3"]["b"])
    g = g @ p["lin4"]["W"] + p["lin4"]["b"]
    return _pl_identity(jax.nn.log_softmax(g, axis=-1))


# speedup vs baseline: 5.2090x; 2.0638x over previous
import functools
import math

import jax
import jax.numpy as jnp
from jax import lax
from jax.experimental import pallas as pl
from jax.experimental.pallas import tpu as pltpu

_B, _N, _K = 4, 4096, 16


def _lrelu(x, s):
    return jnp.where(x >= 0.0, x, s * x)


def _topk_body(pts_ref, qT_ref, qn_ref, idx_ref, val_ref, d2_ref, *, k, nb, flat_base):
    n = pts_ref.shape[1]
    R = qT_ref.shape[2]
    b = pl.program_id(0)
    pts = pts_ref[0]
    pn = jnp.sum(pts * pts, axis=1)
    mm = jnp.dot(pts, qT_ref[0], preferred_element_type=jnp.float32)
    d2_ref[...] = pn[:, None] + qn_ref[0] - 2.0 * mm
    subiota = lax.broadcasted_iota(jnp.int32, (n, R), 0)

    def step(s, _):
        d2c = d2_ref[...]
        mv = jnp.min(d2c, axis=0)
        li = jnp.min(jnp.where(d2c == mv[None, :], subiota, n), axis=0)
        d2_ref[...] = jnp.where(subiota == li[None, :], jnp.inf, d2c)
        val_ref[0, pl.ds(s, 1), :] = mv[None, :]
        gidx = li + b * nb if flat_base else li
        idx_ref[0, pl.ds(s, 1), :] = gidx[None, :]
        return 0

    lax.fori_loop(0, k, step, 0, unroll=False)


def _topk_pallas(q, pts, k, flat_base=True, R=128):
    B, m, _ = q.shape
    n = pts.shape[1]
    mp = ((m + R - 1) // R) * R
    qT = jnp.swapaxes(q, 1, 2)
    qn = jnp.sum(qT * qT, axis=1, keepdims=True)
    if mp != m:
        qT = jnp.pad(qT, ((0, 0), (0, 0), (0, mp - m)))
        qn = jnp.pad(qn, ((0, 0), (0, 0), (0, mp - m)))
    kernel = functools.partial(_topk_body, k=k, nb=n, flat_base=flat_base)
    idx, val = pl.pallas_call(
        kernel,
        grid=(B, mp // R),
        in_specs=[
            pl.BlockSpec((1, n, 3), lambda b, j: (b, 0, 0)),
            pl.BlockSpec((1, 3, R), lambda b, j: (b, 0, j)),
            pl.BlockSpec((1, 1, R), lambda b, j: (b, 0, j)),
        ],
        out_specs=[
            pl.BlockSpec((1, k, R), lambda b, j: (b, 0, j)),
            pl.BlockSpec((1, k, R), lambda b, j: (b, 0, j)),
        ],
        out_shape=[
            jax.ShapeDtypeStruct((B, k, mp), jnp.int32),
            jax.ShapeDtypeStruct((B, k, mp), jnp.float32),
        ],
        scratch_shapes=[pltpu.VMEM((n, R), jnp.float32)],
    )(pts, qT, qn)
    return idx[:, :, :m], val[:, :, :m]


def _fps_body(posT_ref, qpos_ref, dd_ref):
    B, _, n = posT_ref.shape
    m = qpos_ref.shape[1]
    px = posT_ref[:, 0, :]
    py = posT_ref[:, 1, :]
    pz = posT_ref[:, 2, :]
    lanes = lax.broadcasted_iota(jnp.int32, (B, n), 1)

    def extract(j):
        sel = lanes == j[:, None]
        lx = jnp.sum(jnp.where(sel, px, 0.0), axis=1)
        ly = jnp.sum(jnp.where(sel, py, 0.0), axis=1)
        lz = jnp.sum(jnp.where(sel, pz, 0.0), axis=1)
        return lx, ly, lz

    def store_q(i, lx, ly, lz):
        row = jnp.concatenate(
            [lx[:, None, None], ly[:, None, None], lz[:, None, None]], axis=2)
        qpos_ref[:, pl.ds(i, 1), :] = row

    dd_ref[...] = jnp.full((B, n), jnp.inf, jnp.float32)
    j0 = jnp.zeros((B,), jnp.int32)
    lx, ly, lz = extract(j0)
    store_q(0, lx, ly, lz)

    def body(i, carry):
        lx, ly, lz = carry
        d2 = ((px - lx[:, None]) ** 2 + (py - ly[:, None]) ** 2
              + (pz - lz[:, None]) ** 2)
        dd = jnp.minimum(dd_ref[...], d2)
        dd_ref[...] = dd
        mx = jnp.max(dd, axis=1)
        j = jnp.min(jnp.where(dd == mx[:, None], lanes, n), axis=1).astype(jnp.int32)
        lx, ly, lz = extract(j)
        store_q(i, lx, ly, lz)
        return (lx, ly, lz)

    lax.fori_loop(1, m, body, (lx, ly, lz))


def _fps_pallas(pos, m):
    B, n, _ = pos.shape
    posT = jnp.swapaxes(pos, 1, 2)
    return pl.pallas_call(
        _fps_body,
        out_shape=jax.ShapeDtypeStruct((B, m, 3), jnp.float32),
        scratch_shapes=[pltpu.VMEM((B, n), jnp.float32)],
    )(posT)


def _enc_pack(p):
    def lse_parts(lp):
        W = lp["W"]
        aW = W[0:3] + W[6:9]
        nW = W[3:6] - W[6:9]
        dw = W[9]
        return aW, nW, dw

    fcW = p["fc_start"]["W"] * p["bn_start"]["g"][None, :]
    fcb = p["fc_start"]["b"] * p["bn_start"]["g"] + p["bn_start"]["be"]
    a1W, n1W, d1w = lse_parts(p["lse1"])
    a2W, n2W, d2w = lse_parts(p["lse2"])

    def pad32(a):
        a = jnp.asarray(a, jnp.float32)
        if a.ndim == 1:
            a = a[None, :]
        return jnp.pad(a, ((0, 0), (0, 32 - a.shape[1])))

    mats = [
        fcW,
        p["mlp1"]["W"],
        a1W, n1W,
        a2W, n2W,
        p["pool1_score"],
        p["pool2_score"],
        p["pool1_mlp"]["W"],
        p["pool2_mlp"]["W"],
        p["mlp2"]["W"],
        p["shortcut"]["W"],
        fcb,
        p["mlp1"]["b"],
        p["lse1"]["b"], p["lse1"]["g"], p["lse1"]["be"], d1w,
        p["lse2"]["b"], p["lse2"]["g"], p["lse2"]["be"], d2w,
        p["pool1_mlp"]["b"], p["pool1_mlp"]["g"], p["pool1_mlp"]["be"],
        p["pool2_mlp"]["b"], p["pool2_mlp"]["g"], p["pool2_mlp"]["be"],
        p["mlp2"]["b"],
        p["shortcut"]["b"], p["shortcut"]["g"], p["shortcut"]["be"],
    ]
    return jnp.concatenate([pad32(a) for a in mats], axis=0)


def _encoder_body(data_ref, nc_ref, val_ref, w_ref, out_ref):
    P = data_ref.shape[1]
    K = _K
    w = w_ref[...]
    d = data_ref[0]
    ctr = d[:, 0:3]
    loc = d[:, 3:9]
    x0 = _lrelu(jnp.dot(loc, w[0:6, 0:8], preferred_element_type=jnp.float32)
                + w[114, 0:8][None, :], 0.2)
    f1 = _lrelu(jnp.dot(x0, w[6:14, 0:8], preferred_element_type=jnp.float32)
                + w[115, 0:8][None, :], 0.2)

    dv = jnp.maximum(val_ref[0], 0.0)
    dist = jnp.where(dv > 1e-12, jnp.sqrt(jnp.where(dv > 1e-12, dv, 1.0)), 0.0)

    nc = nc_ref[0][:, 0:3]

    def stage(arow, nrow, vrow, f, wsrow, mprow, mpvrow, odim):
        a = jnp.dot(ctr, w[arow:arow + 3, 0:8],
                    preferred_element_type=jnp.float32) + w[vrow, 0:8][None, :]
        ncon = jnp.dot(nc, w[nrow:nrow + 3, 0:8],
                       preferred_element_type=jnp.float32).reshape(P, K, 8)
        pre = a[:, None, :] + ncon + dist[:, :, None] * w[vrow + 3, 0:8][None, None, :]
        enc = (jax.nn.relu(pre) * w[vrow + 1, 0:8][None, None, :]
               + w[vrow + 2, 0:8][None, None, :])
        sp = jnp.dot(enc.reshape(P * K, 8), w[wsrow:wsrow + 8, 0:16],
                     preferred_element_type=jnp.float32).reshape(P, K, 16)
        spf = jnp.dot(f, w[wsrow + 8:wsrow + 16, 0:16],
                      preferred_element_type=jnp.float32)
        s = sp + spf[:, None, :]
        s = s - jnp.max(s, axis=1, keepdims=True)
        es = jnp.exp(s)
        sm = es / jnp.sum(es, axis=1, keepdims=True)
        pe = jnp.sum(sm[:, :, 0:8] * enc, axis=1)
        pf = f * jnp.sum(sm[:, :, 8:16], axis=1)
        pooled = jnp.concatenate([pe, pf], axis=1)
        o = jax.nn.relu(jnp.dot(pooled, w[mprow:mprow + 16, 0:odim],
                                preferred_element_type=jnp.float32)
                        + w[mpvrow, 0:odim][None, :])
        return o * w[mpvrow + 1, 0:odim][None, :] + w[mpvrow + 2, 0:odim][None, :]

    feat2 = stage(14, 17, 116, f1, 26, 58, 124, 8)
    out16 = stage(20, 23, 120, feat2, 42, 74, 127, 16)

    sc = (jnp.dot(x0, w[106:114, 0:32], preferred_element_type=jnp.float32)
          + w[131, 0:32][None, :]) * w[132, 0:32][None, :] + w[133, 0:32][None, :]
    comb = _lrelu(jnp.dot(out16, w[90:106, 0:32], preferred_element_type=jnp.float32)
                  + w[130, 0:32][None, :] + sc, 0.01)
    out_ref[0] = comb


def _encoder_pallas(data, nc, valT, wenc, P=512):
    B, N, _ = data.shape
    return pl.pallas_call(
        _encoder_body,
        grid=(B, N // P),
        in_specs=[
            pl.BlockSpec((1, P, 9), lambda b, j: (b, j, 0)),
            pl.BlockSpec((1, P * _K, 16), lambda b, j: (b, j, 0)),
            pl.BlockSpec((1, P, _K), lambda b, j: (b, j, 0)),
            pl.BlockSpec(wenc.shape, lambda b, j: (0, 0)),
        ],
        out_specs=pl.BlockSpec((1, P, 32), lambda b, j: (b, j, 0)),
        out_shape=jax.ShapeDtypeStruct((B, N, 32), jnp.float32),
    )(data, nc, valT, wenc)


def _samlp_body(g_ref, qrep_ref, vm_ref, w1_ref, w2_ref, w3_ref, vec_ref, out_ref,
                *, C, G):
    rows = g_ref[0]
    xj = rows[:, 0:C]
    pj = rows[:, C:C + 3]
    q = qrep_ref[0]
    dp = pj - q
    w1 = w1_ref[...]
    c1 = w1.shape[1]
    c2 = w2_ref.shape[1]
    c3 = w3_ref.shape[1]
    v = vec_ref[...]
    h = (jnp.dot(xj, w1[0:C], preferred_element_type=jnp.float32)
         + jnp.dot(dp, w1[C:C + 3], preferred_element_type=jnp.float32)
         + v[0, 0:c1][None, :])
    h = jax.nn.relu(h) * v[1, 0:c1][None, :] + v[2, 0:c1][None, :]
    h = jnp.dot(h, w2_ref[...], preferred_element_type=jnp.float32) + v[3, 0:c2][None, :]
    h = jax.nn.relu(h) * v[4, 0:c2][None, :] + v[5, 0:c2][None, :]
    h = jnp.dot(h, w3_ref[...], preferred_element_type=jnp.float32) + v[6, 0:c3][None, :]
    h = jax.nn.relu(h) * v[7, 0:c3][None, :] + v[8, 0:c3][None, :]
    h = jnp.where(vm_ref[0] > 0.0, h, -1e30)
    out_ref[0] = jnp.max(h.reshape(G, 64, c3), axis=1)


def _samlp_pallas(gath, qrep, vmask, layers, C, m, G=64):
    B = gath.shape[0]
    mp = ((m + G - 1) // G) * G
    if mp != m:
        padr = ((0, 0), (0, (mp - m) * 64), (0, 0))
        gath = jnp.pad(gath, padr)
        qrep = jnp.pad(qrep, padr)
        vmask = jnp.pad(vmask, padr)
    c3 = layers[2]["W"].shape[1]
    maxc = max(layers[0]["W"].shape[1], layers[1]["W"].shape[1], c3)

    def padv(a):
        return jnp.pad(a, (0, maxc - a.shape[0]))[None, :]

    vec = jnp.concatenate(
        [padv(layers[i][k]) for i in range(3) for k in ("b", "g", "be")], axis=0)
    Cpad = gath.shape[2]
    kernel = functools.partial(_samlp_body, C=C, G=G)
    out = pl.pallas_call(
        kernel,
        grid=(B, mp // G),
        in_specs=[
            pl.BlockSpec((1, G * 64, Cpad), lambda b, j: (b, j, 0)),
            pl.BlockSpec((1, G * 64, 3), lambda b, j: (b, j, 0)),
            pl.BlockSpec((1, G * 64, 1), lambda b, j: (b, j, 0)),
            pl.BlockSpec(layers[0]["W"].shape, lambda b, j: (0, 0)),
            pl.BlockSpec(layers[1]["W"].shape, lambda b, j: (0, 0)),
            pl.BlockSpec(layers[2]["W"].shape, lambda b, j: (0, 0)),
            pl.BlockSpec(vec.shape, lambda b, j: (0, 0)),
        ],
        out_specs=pl.BlockSpec((1, G, c3), lambda b, j: (b, j, 0)),
        out_shape=jax.ShapeDtypeStruct((B, mp, c3), jnp.float32),
    )(gath, qrep, vmask, layers[0]["W"], layers[1]["W"], layers[2]["W"], vec)
    return out[:, :m]


def _sa_module(x, pos, ratio, r, layers):
    B, n, C = x.shape
    m = max(1, int(n * ratio))
    qpos = _fps_pallas(pos, m)
    nidx, nval = _topk_pallas(qpos, pos, 64, flat_base=True)
    idxT = jnp.swapaxes(nidx, 1, 2).reshape(B, m * 64)
    vmask = (jnp.swapaxes(nval, 1, 2) <= r * r).astype(jnp.float32).reshape(B, m * 64, 1)
    Cpad = ((C + 3 + 15) // 16) * 16
    table = jnp.concatenate(
        [x, pos, jnp.zeros((B, n, Cpad - C - 3), jnp.float32)], axis=-1)
    gath = table.reshape(B * n, Cpad)[idxT.reshape(-1)].reshape(B, m * 64, Cpad)
    qrep = jnp.broadcast_to(qpos[:, :, None, :], (B, m, 64, 3)).reshape(B, m * 64, 3)
    out = _samlp_pallas(gath, qrep, vmask, layers, C, m)
    return out, qpos


def _head_body(x_ref, w1, w2, w3, l1, l2, l3, l4, vec_ref, out_ref):
    v = vec_ref[...]
    h = x_ref[...]
    h = (jnp.dot(h, w1[...], preferred_element_type=jnp.float32) + v[0, 0:256][None, :])
    h = jax.nn.relu(h) * v[1, 0:256][None, :] + v[2, 0:256][None, :]
    h = (jnp.dot(h, w2[...], preferred_element_type=jnp.float32) + v[3, 0:512][None, :])
    h = jax.nn.relu(h) * v[4, 0:512][None, :] + v[5, 0:512][None, :]
    h = (jnp.dot(h, w3[...], preferred_element_type=jnp.float32) + v[6, 0:1024][None, :])
    h = jax.nn.relu(h) * v[7, 0:1024][None, :] + v[8, 0:1024][None, :]
    gs = [jnp.max(h[i * 102:(i + 1) * 102], axis=0, keepdims=True) for i in range(4)]
    g = jnp.concatenate(gs, axis=0)
    g = jax.nn.relu(jnp.dot(g, l1[...], preferred_element_type=jnp.float32)
                    + v[9, 0:512][None, :])
    g = jax.nn.relu(jnp.dot(g, l2[...], preferred_element_type=jnp.float32)
                    + v[10, 0:256][None, :])
    g = jax.nn.relu(jnp.dot(g, l3[...], preferred_element_type=jnp.float32)
                    + v[11, 0:128][None, :])
    g = jnp.dot(g, l4[...], preferred_element_type=jnp.float32) + v[12, 0:2][None, :]
    mx = jnp.max(g, axis=1, keepdims=True)
    sh = g - mx
    out_ref[...] = sh - jnp.log(jnp.sum(jnp.exp(sh), axis=1, keepdims=True))


def _head_pallas(xcat, p):
    sa3 = p["sa3"]

    def padv(a, n=1024):
        return jnp.pad(a, (0, n - a.shape[0]))[None, :]

    vec = jnp.concatenate(
        [padv(sa3[i][k]) for i in range(3) for k in ("b", "g", "be")]
        + [padv(p["lin1"]["b"]), padv(p["lin2"]["b"]), padv(p["lin3"]["b"]),
           padv(p["lin4"]["b"])], axis=0)
    full = lambda a: pl.BlockSpec(a.shape, lambda: (0,) * a.ndim)
    args = (xcat, sa3[0]["W"], sa3[1]["W"], sa3[2]["W"],
            p["lin1"]["W"], p["lin2"]["W"], p["lin3"]["W"], p["lin4"]["W"], vec)
    return pl.pallas_call(
        _head_body,
        in_specs=[full(a) for a in args],
        out_specs=pl.BlockSpec((4, 2), lambda: (0, 0)),
        out_shape=jax.ShapeDtypeStruct((4, 2), jnp.float32),
    )(*args)


def kernel(data, params):
    p = params
    B, N = _B, _N
    coords = data[..., :3]
    knn_idx, knn_val = _topk_pallas(coords, coords, _K, flat_base=True)
    idxT = jnp.swapaxes(knn_idx, 1, 2).reshape(B, N * _K)
    valT = jnp.swapaxes(knn_val, 1, 2)
    ctable = jnp.pad(coords.reshape(B * N, 3), ((0, 0), (0, 13)))
    nc = ctable[idxT.reshape(-1)].reshape(B, N * _K, 16)
    wenc = _enc_pack(p)
    x = _encoder_pallas(data, nc, valT, wenc)
    x = jnp.reshape(jnp.swapaxes(x, 1, 2), (B, N, 32))
    x, pos = _sa_module(x, coords, 0.2, 0.2, p["sa1"])
    x, pos = _sa_module(x, pos, 0.5, 0.2, p["sa1a"])
    x, pos = _sa_module(x, pos, 0.25, 0.4, p["sa2"])
    xcat = jnp.concatenate([x, pos], -1).reshape(B * 102, 259)
    return _head_pallas(xcat, p)


# speedup vs baseline: 6.1755x; 1.1855x over previous
import functools
import math

import jax
import jax.numpy as jnp
from jax import lax
from jax.experimental import pallas as pl
from jax.experimental.pallas import tpu as pltpu
from jax.experimental.pallas import tpu_sc as plsc

_B, _N, _K = 4, 4096, 16


@functools.lru_cache(maxsize=None)
def _make_sc_gather(V, D, Brows):
    NW = 32
    CH = 128
    bpw = Brows // NW
    assert Brows % NW == 0 and bpw % 8 == 0
    nfull, tail = divmod(bpw, CH)
    mesh = plsc.VectorSubcoreMesh(core_axis_name="c", subcore_axis_name="s")

    @functools.partial(
        pl.kernel, mesh=mesh,
        out_type=jax.ShapeDtypeStruct((Brows, D), jnp.float32),
        compiler_params=pltpu.CompilerParams(use_tc_tiling_on_sc=False),
        scratch_types=[
            pltpu.VMEM((CH,), jnp.int32),
            pltpu.VMEM((CH, D), jnp.float32),
            pltpu.SemaphoreType.DMA,
        ],
    )
    def k(table_hbm, idx_hbm, out_hbm, idx_v, rows_v, sem):
        wid = lax.axis_index("s") * 2 + lax.axis_index("c")
        base = wid * bpw

        def do_chunk(off, sz):
            pltpu.sync_copy(idx_hbm.at[pl.ds(off, sz)], idx_v.at[pl.ds(0, sz)])
            pltpu.async_copy(table_hbm.at[idx_v.at[pl.ds(0, sz)]],
                             rows_v.at[pl.ds(0, sz)], sem).wait()
            pltpu.sync_copy(rows_v.at[pl.ds(0, sz)], out_hbm.at[pl.ds(off, sz)])

        def body(i, _):
            do_chunk(base + i * CH, CH)
            return 0

        lax.fori_loop(0, nfull, body, 0)
        if tail:
            do_chunk(base + nfull * CH, tail)

    return k


def _sc_gather(table, idx):
    V, D = table.shape
    return _make_sc_gather(V, D, idx.shape[0])(table, idx)


def _lrelu(x, s):
    return jnp.where(x >= 0.0, x, s * x)


def _topk_body(pts_ref, qT_ref, qn_ref, idx_ref, val_ref, d2_ref, *, k, nb, flat_base):
    n = pts_ref.shape[1]
    R = qT_ref.shape[2]
    b = pl.program_id(0)
    pts = pts_ref[0]
    pn = jnp.sum(pts * pts, axis=1)
    mm = jnp.dot(pts, qT_ref[0], preferred_element_type=jnp.float32)
    d2_ref[...] = pn[:, None] + qn_ref[0] - 2.0 * mm
    subiota = lax.broadcasted_iota(jnp.int32, (n, R), 0)

    def step(s, _):
        d2c = d2_ref[...]
        mv = jnp.min(d2c, axis=0)
        li = jnp.min(jnp.where(d2c == mv[None, :], subiota, n), axis=0)
        d2_ref[...] = jnp.where(subiota == li[None, :], jnp.inf, d2c)
        val_ref[0, pl.ds(s, 1), :] = mv[None, :]
        gidx = li + b * nb if flat_base else li
        idx_ref[0, pl.ds(s, 1), :] = gidx[None, :]
        return 0

    lax.fori_loop(0, k, step, 0, unroll=False)


def _topk_pallas(q, pts, k, flat_base=True, R=128):
    B, m, _ = q.shape
    n = pts.shape[1]
    mp = ((m + R - 1) // R) * R
    qT = jnp.swapaxes(q, 1, 2)
    qn = jnp.sum(qT * qT, axis=1, keepdims=True)
    if mp != m:
        qT = jnp.pad(qT, ((0, 0), (0, 0), (0, mp - m)))
        qn = jnp.pad(qn, ((0, 0), (0, 0), (0, mp - m)))
    kernel = functools.partial(_topk_body, k=k, nb=n, flat_base=flat_base)
    idx, val = pl.pallas_call(
        kernel,
        grid=(B, mp // R),
        in_specs=[
            pl.BlockSpec((1, n, 3), lambda b, j: (b, 0, 0)),
            pl.BlockSpec((1, 3, R), lambda b, j: (b, 0, j)),
            pl.BlockSpec((1, 1, R), lambda b, j: (b, 0, j)),
        ],
        out_specs=[
            pl.BlockSpec((1, k, R), lambda b, j: (b, 0, j)),
            pl.BlockSpec((1, k, R), lambda b, j: (b, 0, j)),
        ],
        out_shape=[
            jax.ShapeDtypeStruct((B, k, mp), jnp.int32),
            jax.ShapeDtypeStruct((B, k, mp), jnp.float32),
        ],
        scratch_shapes=[pltpu.VMEM((n, R), jnp.float32)],
    )(pts, qT, qn)
    return idx[:, :, :m], val[:, :, :m]


def _fps_body(posT_ref, qpos_ref, dd_ref):
    B, _, n = posT_ref.shape
    m = qpos_ref.shape[1]
    px = posT_ref[:, 0, :]
    py = posT_ref[:, 1, :]
    pz = posT_ref[:, 2, :]
    lanes = lax.broadcasted_iota(jnp.int32, (B, n), 1)

    def extract(j):
        sel = lanes == j[:, None]
        lx = jnp.sum(jnp.where(sel, px, 0.0), axis=1)
        ly = jnp.sum(jnp.where(sel, py, 0.0), axis=1)
        lz = jnp.sum(jnp.where(sel, pz, 0.0), axis=1)
        return lx, ly, lz

    def store_q(i, lx, ly, lz):
        row = jnp.concatenate(
            [lx[:, None, None], ly[:, None, None], lz[:, None, None]], axis=2)
        qpos_ref[:, pl.ds(i, 1), :] = row

    dd_ref[...] = jnp.full((B, n), jnp.inf, jnp.float32)
    j0 = jnp.zeros((B,), jnp.int32)
    lx, ly, lz = extract(j0)
    store_q(0, lx, ly, lz)

    def body(i, carry):
        lx, ly, lz = carry
        d2 = ((px - lx[:, None]) ** 2 + (py - ly[:, None]) ** 2
              + (pz - lz[:, None]) ** 2)
        dd = jnp.minimum(dd_ref[...], d2)
        dd_ref[...] = dd
        mx = jnp.max(dd, axis=1)
        j = jnp.min(jnp.where(dd == mx[:, None], lanes, n), axis=1).astype(jnp.int32)
        lx, ly, lz = extract(j)
        store_q(i, lx, ly, lz)
        return (lx, ly, lz)

    lax.fori_loop(1, m, body, (lx, ly, lz))


def _fps_pallas(pos, m):
    B, n, _ = pos.shape
    posT = jnp.swapaxes(pos, 1, 2)
    return pl.pallas_call(
        _fps_body,
        out_shape=jax.ShapeDtypeStruct((B, m, 3), jnp.float32),
        scratch_shapes=[pltpu.VMEM((B, n), jnp.float32)],
    )(posT)


def _enc_pack(p):
    def lse_parts(lp):
        W = lp["W"]
        aW = W[0:3] + W[6:9]
        nW = W[3:6] - W[6:9]
        dw = W[9]
        return aW, nW, dw

    fcW = p["fc_start"]["W"] * p["bn_start"]["g"][None, :]
    fcb = p["fc_start"]["b"] * p["bn_start"]["g"] + p["bn_start"]["be"]
    a1W, n1W, d1w = lse_parts(p["lse1"])
    a2W, n2W, d2w = lse_parts(p["lse2"])

    def pad32(a):
        a = jnp.asarray(a, jnp.float32)
        if a.ndim == 1:
            a = a[None, :]
        return jnp.pad(a, ((0, 0), (0, 32 - a.shape[1])))

    mats = [
        fcW,
        p["mlp1"]["W"],
        a1W, n1W,
        a2W, n2W,
        p["pool1_score"],
        p["pool2_score"],
        p["pool1_mlp"]["W"],
        p["pool2_mlp"]["W"],
        p["mlp2"]["W"],
        p["shortcut"]["W"],
        fcb,
        p["mlp1"]["b"],
        p["lse1"]["b"], p["lse1"]["g"], p["lse1"]["be"], d1w,
        p["lse2"]["b"], p["lse2"]["g"], p["lse2"]["be"], d2w,
        p["pool1_mlp"]["b"], p["pool1_mlp"]["g"], p["pool1_mlp"]["be"],
        p["pool2_mlp"]["b"], p["pool2_mlp"]["g"], p["pool2_mlp"]["be"],
        p["mlp2"]["b"],
        p["shortcut"]["b"], p["shortcut"]["g"], p["shortcut"]["be"],
    ]
    return jnp.concatenate([pad32(a) for a in mats], axis=0)


def _encoder_body(data_ref, nc_ref, val_ref, w_ref, out_ref):
    P = data_ref.shape[1]
    K = _K
    w = w_ref[...]
    d = data_ref[0]
    ctr = d[:, 0:3]
    loc = d[:, 3:9]
    x0 = _lrelu(jnp.dot(loc, w[0:6, 0:8], preferred_element_type=jnp.float32)
                + w[114, 0:8][None, :], 0.2)
    f1 = _lrelu(jnp.dot(x0, w[6:14, 0:8], preferred_element_type=jnp.float32)
                + w[115, 0:8][None, :], 0.2)

    dv = jnp.maximum(val_ref[0], 0.0)
    dist = jnp.where(dv > 1e-12, jnp.sqrt(jnp.where(dv > 1e-12, dv, 1.0)), 0.0)

    nc = nc_ref[0][:, 0:3]

    def stage(arow, nrow, vrow, f, wsrow, mprow, mpvrow, odim):
        a = jnp.dot(ctr, w[arow:arow + 3, 0:8],
                    preferred_element_type=jnp.float32) + w[vrow, 0:8][None, :]
        ncon = jnp.dot(nc, w[nrow:nrow + 3, 0:8],
                       preferred_element_type=jnp.float32).reshape(P, K, 8)
        pre = a[:, None, :] + ncon + dist[:, :, None] * w[vrow + 3, 0:8][None, None, :]
        enc = (jax.nn.relu(pre) * w[vrow + 1, 0:8][None, None, :]
               + w[vrow + 2, 0:8][None, None, :])
        sp = jnp.dot(enc.reshape(P * K, 8), w[wsrow:wsrow + 8, 0:16],
                     preferred_element_type=jnp.float32).reshape(P, K, 16)
        spf = jnp.dot(f, w[wsrow + 8:wsrow + 16, 0:16],
                      preferred_element_type=jnp.float32)
        s = sp + spf[:, None, :]
        s = s - jnp.max(s, axis=1, keepdims=True)
        es = jnp.exp(s)
        sm = es / jnp.sum(es, axis=1, keepdims=True)
        pe = jnp.sum(sm[:, :, 0:8] * enc, axis=1)
        pf = f * jnp.sum(sm[:, :, 8:16], axis=1)
        pooled = jnp.concatenate([pe, pf], axis=1)
        o = jax.nn.relu(jnp.dot(pooled, w[mprow:mprow + 16, 0:odim],
                                preferred_element_type=jnp.float32)
                        + w[mpvrow, 0:odim][None, :])
        return o * w[mpvrow + 1, 0:odim][None, :] + w[mpvrow + 2, 0:odim][None, :]

    feat2 = stage(14, 17, 116, f1, 26, 58, 124, 8)
    out16 = stage(20, 23, 120, feat2, 42, 74, 127, 16)

    sc = (jnp.dot(x0, w[106:114, 0:32], preferred_element_type=jnp.float32)
          + w[131, 0:32][None, :]) * w[132, 0:32][None, :] + w[133, 0:32][None, :]
    comb = _lrelu(jnp.dot(out16, w[90:106, 0:32], preferred_element_type=jnp.float32)
                  + w[130, 0:32][None, :] + sc, 0.01)
    out_ref[0] = comb


def _encoder_pallas(data, nc, valT, wenc, P=512):
    B, N, _ = data.shape
    return pl.pallas_call(
        _encoder_body,
        grid=(B, N // P),
        in_specs=[
            pl.BlockSpec((1, P, 9), lambda b, j: (b, j, 0)),
            pl.BlockSpec((1, P * _K, 16), lambda b, j: (b, j, 0)),
            pl.BlockSpec((1, P, _K), lambda b, j: (b, j, 0)),
            pl.BlockSpec(wenc.shape, lambda b, j: (0, 0)),
        ],
        out_specs=pl.BlockSpec((1, P, 32), lambda b, j: (b, j, 0)),
        out_shape=jax.ShapeDtypeStruct((B, N, 32), jnp.float32),
    )(data, nc, valT, wenc)


def _samlp_body(g_ref, qrep_ref, vm_ref, w1_ref, w2_ref, w3_ref, vec_ref, out_ref,
                *, C, G):
    rows = g_ref[0]
    xj = rows[:, 0:C]
    pj = rows[:, C:C + 3]
    q = qrep_ref[0]
    dp = pj - q
    w1 = w1_ref[...]
    c1 = w1.shape[1]
    c2 = w2_ref.shape[1]
    c3 = w3_ref.shape[1]
    v = vec_ref[...]
    h = (jnp.dot(xj, w1[0:C], preferred_element_type=jnp.float32)
         + jnp.dot(dp, w1[C:C + 3], preferred_element_type=jnp.float32)
         + v[0, 0:c1][None, :])
    h = jax.nn.relu(h) * v[1, 0:c1][None, :] + v[2, 0:c1][None, :]
    h = jnp.dot(h, w2_ref[...], preferred_element_type=jnp.float32) + v[3, 0:c2][None, :]
    h = jax.nn.relu(h) * v[4, 0:c2][None, :] + v[5, 0:c2][None, :]
    h = jnp.dot(h, w3_ref[...], preferred_element_type=jnp.float32) + v[6, 0:c3][None, :]
    h = jax.nn.relu(h) * v[7, 0:c3][None, :] + v[8, 0:c3][None, :]
    h = jnp.where(vm_ref[0] > 0.0, h, -1e30)
    out_ref[0] = jnp.max(h.reshape(G, 64, c3), axis=1)


def _samlp_pallas(gath, qrep, vmask, layers, C, m, G=64):
    B = gath.shape[0]
    mp = ((m + G - 1) // G) * G
    if mp != m:
        padr = ((0, 0), (0, (mp - m) * 64), (0, 0))
        gath = jnp.pad(gath, padr)
        qrep = jnp.pad(qrep, padr)
        vmask = jnp.pad(vmask, padr)
    c3 = layers[2]["W"].shape[1]
    maxc = max(layers[0]["W"].shape[1], layers[1]["W"].shape[1], c3)

    def padv(a):
        return jnp.pad(a, (0, maxc - a.shape[0]))[None, :]

    vec = jnp.concatenate(
        [padv(layers[i][k]) for i in range(3) for k in ("b", "g", "be")], axis=0)
    Cpad = gath.shape[2]
    kernel = functools.partial(_samlp_body, C=C, G=G)
    out = pl.pallas_call(
        kernel,
        grid=(B, mp // G),
        in_specs=[
            pl.BlockSpec((1, G * 64, Cpad), lambda b, j: (b, j, 0)),
            pl.BlockSpec((1, G * 64, 3), lambda b, j: (b, j, 0)),
            pl.BlockSpec((1, G * 64, 1), lambda b, j: (b, j, 0)),
            pl.BlockSpec(layers[0]["W"].shape, lambda b, j: (0, 0)),
            pl.BlockSpec(layers[1]["W"].shape, lambda b, j: (0, 0)),
            pl.BlockSpec(layers[2]["W"].shape, lambda b, j: (0, 0)),
            pl.BlockSpec(vec.shape, lambda b, j: (0, 0)),
        ],
        out_specs=pl.BlockSpec((1, G, c3), lambda b, j: (b, j, 0)),
        out_shape=jax.ShapeDtypeStruct((B, mp, c3), jnp.float32),
    )(gath, qrep, vmask, layers[0]["W"], layers[1]["W"], layers[2]["W"], vec)
    return out[:, :m]


def _sa_module(x, pos, ratio, r, layers):
    B, n, C = x.shape
    m = max(1, int(n * ratio))
    qpos = _fps_pallas(pos, m)
    nidx, nval = _topk_pallas(qpos, pos, 64, flat_base=True)
    idxT = jnp.swapaxes(nidx, 1, 2).reshape(B, m * 64)
    vmask = (jnp.swapaxes(nval, 1, 2) <= r * r).astype(jnp.float32).reshape(B, m * 64, 1)
    Cpad = ((C + 3 + 15) // 16) * 16
    table = jnp.concatenate(
        [x, pos, jnp.zeros((B, n, Cpad - C - 3), jnp.float32)], axis=-1)
    gath = _sc_gather(table.reshape(B * n, Cpad), idxT.reshape(-1)).reshape(B, m * 64, Cpad)
    qrep = jnp.broadcast_to(qpos[:, :, None, :], (B, m, 64, 3)).reshape(B, m * 64, 3)
    out = _samlp_pallas(gath, qrep, vmask, layers, C, m)
    return out, qpos


def _head_body(x_ref, w1, w2, w3, l1, l2, l3, l4, vec_ref, out_ref):
    v = vec_ref[...]
    h = x_ref[...]
    h = (jnp.dot(h, w1[...], preferred_element_type=jnp.float32) + v[0, 0:256][None, :])
    h = jax.nn.relu(h) * v[1, 0:256][None, :] + v[2, 0:256][None, :]
    h = (jnp.dot(h, w2[...], preferred_element_type=jnp.float32) + v[3, 0:512][None, :])
    h = jax.nn.relu(h) * v[4, 0:512][None, :] + v[5, 0:512][None, :]
    h = (jnp.dot(h, w3[...], preferred_element_type=jnp.float32) + v[6, 0:1024][None, :])
    h = jax.nn.relu(h) * v[7, 0:1024][None, :] + v[8, 0:1024][None, :]
    gs = [jnp.max(h[i * 102:(i + 1) * 102], axis=0, keepdims=True) for i in range(4)]
    g = jnp.concatenate(gs, axis=0)
    g = jax.nn.relu(jnp.dot(g, l1[...], preferred_element_type=jnp.float32)
                    + v[9, 0:512][None, :])
    g = jax.nn.relu(jnp.dot(g, l2[...], preferred_element_type=jnp.float32)
                    + v[10, 0:256][None, :])
    g = jax.nn.relu(jnp.dot(g, l3[...], preferred_element_type=jnp.float32)
                    + v[11, 0:128][None, :])
    g = jnp.dot(g, l4[...], preferred_element_type=jnp.float32) + v[12, 0:2][None, :]
    mx = jnp.max(g, axis=1, keepdims=True)
    sh = g - mx
    out_ref[...] = sh - jnp.log(jnp.sum(jnp.exp(sh), axis=1, keepdims=True))


def _head_pallas(xcat, p):
    sa3 = p["sa3"]

    def padv(a, n=1024):
        return jnp.pad(a, (0, n - a.shape[0]))[None, :]

    vec = jnp.concatenate(
        [padv(sa3[i][k]) for i in range(3) for k in ("b", "g", "be")]
        + [padv(p["lin1"]["b"]), padv(p["lin2"]["b"]), padv(p["lin3"]["b"]),
           padv(p["lin4"]["b"])], axis=0)
    full = lambda a: pl.BlockSpec(a.shape, lambda: (0,) * a.ndim)
    args = (xcat, sa3[0]["W"], sa3[1]["W"], sa3[2]["W"],
            p["lin1"]["W"], p["lin2"]["W"], p["lin3"]["W"], p["lin4"]["W"], vec)
    return pl.pallas_call(
        _head_body,
        in_specs=[full(a) for a in args],
        out_specs=pl.BlockSpec((4, 2), lambda: (0, 0)),
        out_shape=jax.ShapeDtypeStruct((4, 2), jnp.float32),
    )(*args)


def kernel(data, params):
    p = params
    B, N = _B, _N
    coords = data[..., :3]
    knn_idx, knn_val = _topk_pallas(coords, coords, _K, flat_base=True)
    idxT = jnp.swapaxes(knn_idx, 1, 2).reshape(B, N * _K)
    valT = jnp.swapaxes(knn_val, 1, 2)
    ctable = jnp.pad(coords.reshape(B * N, 3), ((0, 0), (0, 13)))
    nc = _sc_gather(ctable, idxT.reshape(-1)).reshape(B, N * _K, 16)
    wenc = _enc_pack(p)
    x = _encoder_pallas(data, nc, valT, wenc)
    x = jnp.reshape(jnp.swapaxes(x, 1, 2), (B, N, 32))
    x, pos = _sa_module(x, coords, 0.2, 0.2, p["sa1"])
    x, pos = _sa_module(x, pos, 0.5, 0.2, p["sa1a"])
    x, pos = _sa_module(x, pos, 0.25, 0.4, p["sa2"])
    xcat = jnp.concatenate([x, pos], -1).reshape(B * 102, 259)
    return _head_pallas(xcat, p)


# speedup vs baseline: 6.5386x; 1.0588x over previous
import functools
import math

import jax
import jax.numpy as jnp
from jax import lax
from jax.experimental import pallas as pl
from jax.experimental.pallas import tpu as pltpu
from jax.experimental.pallas import tpu_sc as plsc

_B, _N, _K = 4, 4096, 16


@functools.lru_cache(maxsize=None)
def _make_sc_gather(V, D, Brows):
    NW = 32
    CH = 128
    bpw = Brows // NW
    assert Brows % NW == 0 and bpw % 8 == 0
    nfull, tail = divmod(bpw, CH)
    mesh = plsc.VectorSubcoreMesh(core_axis_name="c", subcore_axis_name="s")

    @functools.partial(
        pl.kernel, mesh=mesh,
        out_type=jax.ShapeDtypeStruct((Brows, D), jnp.float32),
        compiler_params=pltpu.CompilerParams(use_tc_tiling_on_sc=False),
        scratch_types=[
            pltpu.VMEM((CH,), jnp.int32),
            pltpu.VMEM((CH, D), jnp.float32),
            pltpu.SemaphoreType.DMA,
        ],
    )
    def k(table_hbm, idx_hbm, out_hbm, idx_v, rows_v, sem):
        wid = lax.axis_index("s") * 2 + lax.axis_index("c")
        base = wid * bpw

        def do_chunk(off, sz):
            pltpu.sync_copy(idx_hbm.at[pl.ds(off, sz)], idx_v.at[pl.ds(0, sz)])
            pltpu.async_copy(table_hbm.at[idx_v.at[pl.ds(0, sz)]],
                             rows_v.at[pl.ds(0, sz)], sem).wait()
            pltpu.sync_copy(rows_v.at[pl.ds(0, sz)], out_hbm.at[pl.ds(off, sz)])

        def body(i, _):
            do_chunk(base + i * CH, CH)
            return 0

        lax.fori_loop(0, nfull, body, 0)
        if tail:
            do_chunk(base + nfull * CH, tail)

    return k


def _sc_gather(table, idx):
    V, D = table.shape
    return _make_sc_gather(V, D, idx.shape[0])(table, idx)


def _lrelu(x, s):
    return jnp.where(x >= 0.0, x, s * x)


def _topk_body(pts_ref, qT_ref, qn_ref, idx_ref, val_ref, d2_ref, *, k, nb,
               flat_base, rmax):
    n = pts_ref.shape[1]
    R = qT_ref.shape[2]
    b = pl.program_id(0)
    pts = pts_ref[0]
    pn = jnp.sum(pts * pts, axis=1)
    mm = jnp.dot(pts, qT_ref[0], preferred_element_type=jnp.float32)
    d2_ref[...] = pn[:, None] + qn_ref[0] - 2.0 * mm
    subiota = lax.broadcasted_iota(jnp.int32, (n, R), 0)

    def step(s):
        d2c = d2_ref[...]
        mv = jnp.min(d2c, axis=0)
        li = jnp.min(jnp.where(d2c == mv[None, :], subiota, n), axis=0)
        d2_ref[...] = jnp.where(subiota == li[None, :], jnp.inf, d2c)
        val_ref[0, pl.ds(s, 1), :] = mv[None, :]
        gidx = li + b * nb if flat_base else li
        idx_ref[0, pl.ds(s, 1), :] = gidx[None, :]
        return mv

    if rmax is None:
        def fbody(s, _):
            step(s)
            return 0
        lax.fori_loop(0, k, fbody, 0, unroll=False)
    else:
        val_ref[0] = jnp.full((k, R), jnp.inf, jnp.float32)
        idx_ref[0] = jnp.full((k, R), b * nb if flat_base else 0, jnp.int32)

        def cond(c):
            s, alive = c
            return jnp.logical_and(alive, s < k)

        def wbody(c):
            s, _ = c
            mv = step(s)
            return (s + 1, jnp.any(mv <= rmax))

        lax.while_loop(cond, wbody, (jnp.int32(0), True))


def _topk_pallas(q, pts, k, flat_base=True, R=128, rmax=None):
    B, m, _ = q.shape
    n = pts.shape[1]
    mp = ((m + R - 1) // R) * R
    qT = jnp.swapaxes(q, 1, 2)
    qn = jnp.sum(qT * qT, axis=1, keepdims=True)
    if mp != m:
        qT = jnp.pad(qT, ((0, 0), (0, 0), (0, mp - m)))
        qn = jnp.pad(qn, ((0, 0), (0, 0), (0, mp - m)), constant_values=1e30)
    kernel = functools.partial(_topk_body, k=k, nb=n, flat_base=flat_base, rmax=rmax)
    idx, val = pl.pallas_call(
        kernel,
        grid=(B, mp // R),
        in_specs=[
            pl.BlockSpec((1, n, 3), lambda b, j: (b, 0, 0)),
            pl.BlockSpec((1, 3, R), lambda b, j: (b, 0, j)),
            pl.BlockSpec((1, 1, R), lambda b, j: (b, 0, j)),
        ],
        out_specs=[
            pl.BlockSpec((1, k, R), lambda b, j: (b, 0, j)),
            pl.BlockSpec((1, k, R), lambda b, j: (b, 0, j)),
        ],
        out_shape=[
            jax.ShapeDtypeStruct((B, k, mp), jnp.int32),
            jax.ShapeDtypeStruct((B, k, mp), jnp.float32),
        ],
        scratch_shapes=[pltpu.VMEM((n, R), jnp.float32)],
    )(pts, qT, qn)
    return idx[:, :, :m], val[:, :, :m]


def _fps_body(posT_ref, qpos_ref, dd_ref):
    B, _, n = posT_ref.shape
    m = qpos_ref.shape[1]
    px = posT_ref[:, 0, :]
    py = posT_ref[:, 1, :]
    pz = posT_ref[:, 2, :]
    lanes = lax.broadcasted_iota(jnp.int32, (B, n), 1)

    def extract(j):
        sel = lanes == j[:, None]
        lx = jnp.sum(jnp.where(sel, px, 0.0), axis=1)
        ly = jnp.sum(jnp.where(sel, py, 0.0), axis=1)
        lz = jnp.sum(jnp.where(sel, pz, 0.0), axis=1)
        return lx, ly, lz

    def store_q(i, lx, ly, lz):
        row = jnp.concatenate(
            [lx[:, None, None], ly[:, None, None], lz[:, None, None]], axis=2)
        qpos_ref[:, pl.ds(i, 1), :] = row

    dd_ref[...] = jnp.full((B, n), jnp.inf, jnp.float32)
    j0 = jnp.zeros((B,), jnp.int32)
    lx, ly, lz = extract(j0)
    store_q(0, lx, ly, lz)

    def body(i, carry):
        lx, ly, lz = carry
        d2 = ((px - lx[:, None]) ** 2 + (py - ly[:, None]) ** 2
              + (pz - lz[:, None]) ** 2)
        dd = jnp.minimum(dd_ref[...], d2)
        dd_ref[...] = dd
        mx = jnp.max(dd, axis=1)
        j = jnp.min(jnp.where(dd == mx[:, None], lanes, n), axis=1).astype(jnp.int32)
        lx, ly, lz = extract(j)
        store_q(i, lx, ly, lz)
        return (lx, ly, lz)

    lax.fori_loop(1, m, body, (lx, ly, lz))


def _fps_pallas(pos, m):
    B, n, _ = pos.shape
    posT = jnp.swapaxes(pos, 1, 2)
    return pl.pallas_call(
        _fps_body,
        out_shape=jax.ShapeDtypeStruct((B, m, 3), jnp.float32),
        scratch_shapes=[pltpu.VMEM((B, n), jnp.float32)],
    )(posT)


def _enc_pack(p):
    def lse_parts(lp):
        W = lp["W"]
        aW = W[0:3] + W[6:9]
        nW = W[3:6] - W[6:9]
        dw = W[9]
        return aW, nW, dw

    fcW = p["fc_start"]["W"] * p["bn_start"]["g"][None, :]
    fcb = p["fc_start"]["b"] * p["bn_start"]["g"] + p["bn_start"]["be"]
    a1W, n1W, d1w = lse_parts(p["lse1"])
    a2W, n2W, d2w = lse_parts(p["lse2"])

    def pad32(a):
        a = jnp.asarray(a, jnp.float32)
        if a.ndim == 1:
            a = a[None, :]
        return jnp.pad(a, ((0, 0), (0, 32 - a.shape[1])))

    mats = [
        fcW,
        p["mlp1"]["W"],
        a1W, n1W,
        a2W, n2W,
        p["pool1_score"],
        p["pool2_score"],
        p["pool1_mlp"]["W"],
        p["pool2_mlp"]["W"],
        p["mlp2"]["W"],
        p["shortcut"]["W"],
        fcb,
        p["mlp1"]["b"],
        p["lse1"]["b"], p["lse1"]["g"], p["lse1"]["be"], d1w,
        p["lse2"]["b"], p["lse2"]["g"], p["lse2"]["be"], d2w,
        p["pool1_mlp"]["b"], p["pool1_mlp"]["g"], p["pool1_mlp"]["be"],
        p["pool2_mlp"]["b"], p["pool2_mlp"]["g"], p["pool2_mlp"]["be"],
        p["mlp2"]["b"],
        p["shortcut"]["b"], p["shortcut"]["g"], p["shortcut"]["be"],
    ]
    return jnp.concatenate([pad32(a) for a in mats], axis=0)


def _encoder_body(data_ref, nc_ref, val_ref, w_ref, out_ref):
    P = data_ref.shape[1]
    K = _K
    w = w_ref[...]
    d = data_ref[0]
    ctr = d[:, 0:3]
    loc = d[:, 3:9]
    x0 = _lrelu(jnp.dot(loc, w[0:6, 0:8], preferred_element_type=jnp.float32)
                + w[114, 0:8][None, :], 0.2)
    f1 = _lrelu(jnp.dot(x0, w[6:14, 0:8], preferred_element_type=jnp.float32)
                + w[115, 0:8][None, :], 0.2)

    dv = jnp.maximum(val_ref[0], 0.0)
    dist = jnp.where(dv > 1e-12, jnp.sqrt(jnp.where(dv > 1e-12, dv, 1.0)), 0.0)

    nc = nc_ref[0][:, 0:3]

    def stage(arow, nrow, vrow, f, wsrow, mprow, mpvrow, odim):
        a = jnp.dot(ctr, w[arow:arow + 3, 0:8],
                    preferred_element_type=jnp.float32) + w[vrow, 0:8][None, :]
        ncon = jnp.dot(nc, w[nrow:nrow + 3, 0:8],
                       preferred_element_type=jnp.float32).reshape(P, K, 8)
        pre = a[:, None, :] + ncon + dist[:, :, None] * w[vrow + 3, 0:8][None, None, :]
        enc = (jax.nn.relu(pre) * w[vrow + 1, 0:8][None, None, :]
               + w[vrow + 2, 0:8][None, None, :])
        sp = jnp.dot(enc.reshape(P * K, 8), w[wsrow:wsrow + 8, 0:16],
                     preferred_element_type=jnp.float32).reshape(P, K, 16)
        spf = jnp.dot(f, w[wsrow + 8:wsrow + 16, 0:16],
                      preferred_element_type=jnp.float32)
        s = sp + spf[:, None, :]
        s = s - jnp.max(s, axis=1, keepdims=True)
        es = jnp.exp(s)
        sm = es / jnp.sum(es, axis=1, keepdims=True)
        pe = jnp.sum(sm[:, :, 0:8] * enc, axis=1)
        pf = f * jnp.sum(sm[:, :, 8:16], axis=1)
        pooled = jnp.concatenate([pe, pf], axis=1)
        o = jax.nn.relu(jnp.dot(pooled, w[mprow:mprow + 16, 0:odim],
                                preferred_element_type=jnp.float32)
                        + w[mpvrow, 0:odim][None, :])
        return o * w[mpvrow + 1, 0:odim][None, :] + w[mpvrow + 2, 0:odim][None, :]

    feat2 = stage(14, 17, 116, f1, 26, 58, 124, 8)
    out16 = stage(20, 23, 120, feat2, 42, 74, 127, 16)

    sc = (jnp.dot(x0, w[106:114, 0:32], preferred_element_type=jnp.float32)
          + w[131, 0:32][None, :]) * w[132, 0:32][None, :] + w[133, 0:32][None, :]
    comb = _lrelu(jnp.dot(out16, w[90:106, 0:32], preferred_element_type=jnp.float32)
                  + w[130, 0:32][None, :] + sc, 0.01)
    out_ref[0] = comb


def _encoder_pallas(data, nc, valT, wenc, P=512):
    B, N, _ = data.shape
    return pl.pallas_call(
        _encoder_body,
        grid=(B, N // P),
        in_specs=[
            pl.BlockSpec((1, P, 9), lambda b, j: (b, j, 0)),
            pl.BlockSpec((1, P * _K, 16), lambda b, j: (b, j, 0)),
            pl.BlockSpec((1, P, _K), lambda b, j: (b, j, 0)),
            pl.BlockSpec(wenc.shape, lambda b, j: (0, 0)),
        ],
        out_specs=pl.BlockSpec((1, P, 32), lambda b, j: (b, j, 0)),
        out_shape=jax.ShapeDtypeStruct((B, N, 32), jnp.float32),
    )(data, nc, valT, wenc)


def _samlp_body(g_ref, qrep_ref, vm_ref, w1_ref, w2_ref, w3_ref, vec_ref, out_ref,
                *, C, G):
    rows = g_ref[0]
    xj = rows[:, 0:C]
    pj = rows[:, C:C + 3]
    q = qrep_ref[0]
    dp = pj - q
    w1 = w1_ref[...]
    c1 = w1.shape[1]
    c2 = w2_ref.shape[1]
    c3 = w3_ref.shape[1]
    v = vec_ref[...]
    h = (jnp.dot(xj, w1[0:C], preferred_element_type=jnp.float32)
         + jnp.dot(dp, w1[C:C + 3], preferred_element_type=jnp.float32)
         + v[0, 0:c1][None, :])
    h = jax.nn.relu(h) * v[1, 0:c1][None, :] + v[2, 0:c1][None, :]
    h = jnp.dot(h, w2_ref[...], preferred_element_type=jnp.float32) + v[3, 0:c2][None, :]
    h = jax.nn.relu(h) * v[4, 0:c2][None, :] + v[5, 0:c2][None, :]
    h = jnp.dot(h, w3_ref[...], preferred_element_type=jnp.float32) + v[6, 0:c3][None, :]
    h = jax.nn.relu(h) * v[7, 0:c3][None, :] + v[8, 0:c3][None, :]
    h = jnp.where(vm_ref[0] > 0.0, h, -1e30)
    out_ref[0] = jnp.max(h.reshape(G, 64, c3), axis=1)


def _samlp_pallas(gath, qrep, vmask, layers, C, m, G=64):
    B = gath.shape[0]
    mp = ((m + G - 1) // G) * G
    if mp != m:
        padr = ((0, 0), (0, (mp - m) * 64), (0, 0))
        gath = jnp.pad(gath, padr)
        qrep = jnp.pad(qrep, padr)
        vmask = jnp.pad(vmask, padr)
    c3 = layers[2]["W"].shape[1]
    maxc = max(layers[0]["W"].shape[1], layers[1]["W"].shape[1], c3)

    def padv(a):
        return jnp.pad(a, (0, maxc - a.shape[0]))[None, :]

    vec = jnp.concatenate(
        [padv(layers[i][k]) for i in range(3) for k in ("b", "g", "be")], axis=0)
    Cpad = gath.shape[2]
    kernel = functools.partial(_samlp_body, C=C, G=G)
    out = pl.pallas_call(
        kernel,
        grid=(B, mp // G),
        in_specs=[
            pl.BlockSpec((1, G * 64, Cpad), lambda b, j: (b, j, 0)),
            pl.BlockSpec((1, G * 64, 3), lambda b, j: (b, j, 0)),
            pl.BlockSpec((1, G * 64, 1), lambda b, j: (b, j, 0)),
            pl.BlockSpec(layers[0]["W"].shape, lambda b, j: (0, 0)),
            pl.BlockSpec(layers[1]["W"].shape, lambda b, j: (0, 0)),
            pl.BlockSpec(layers[2]["W"].shape, lambda b, j: (0, 0)),
            pl.BlockSpec(vec.shape, lambda b, j: (0, 0)),
        ],
        out_specs=pl.BlockSpec((1, G, c3), lambda b, j: (b, j, 0)),
        out_shape=jax.ShapeDtypeStruct((B, mp, c3), jnp.float32),
    )(gath, qrep, vmask, layers[0]["W"], layers[1]["W"], layers[2]["W"], vec)
    return out[:, :m]


def _sa_module(x, pos, ratio, r, layers):
    B, n, C = x.shape
    m = max(1, int(n * ratio))
    qpos = _fps_pallas(pos, m)
    nidx, nval = _topk_pallas(qpos, pos, 64, flat_base=True, rmax=r * r)
    idxT = jnp.swapaxes(nidx, 1, 2).reshape(B, m * 64)
    vmask = (jnp.swapaxes(nval, 1, 2) <= r * r).astype(jnp.float32).reshape(B, m * 64, 1)
    Cpad = ((C + 3 + 15) // 16) * 16
    table = jnp.concatenate(
        [x, pos, jnp.zeros((B, n, Cpad - C - 3), jnp.float32)], axis=-1)
    gath = _sc_gather(table.reshape(B * n, Cpad), idxT.reshape(-1)).reshape(B, m * 64, Cpad)
    qrep = jnp.broadcast_to(qpos[:, :, None, :], (B, m, 64, 3)).reshape(B, m * 64, 3)
    out = _samlp_pallas(gath, qrep, vmask, layers, C, m)
    return out, qpos


def _head_body(x_ref, w1, w2, w3, l1, l2, l3, l4, vec_ref, out_ref):
    v = vec_ref[...]
    h = x_ref[...]
    h = (jnp.dot(h, w1[...], preferred_element_type=jnp.float32) + v[0, 0:256][None, :])
    h = jax.nn.relu(h) * v[1, 0:256][None, :] + v[2, 0:256][None, :]
    h = (jnp.dot(h, w2[...], preferred_element_type=jnp.float32) + v[3, 0:512][None, :])
    h = jax.nn.relu(h) * v[4, 0:512][None, :] + v[5, 0:512][None, :]
    h = (jnp.dot(h, w3[...], preferred_element_type=jnp.float32) + v[6, 0:1024][None, :])
    h = jax.nn.relu(h) * v[7, 0:1024][None, :] + v[8, 0:1024][None, :]
    gs = [jnp.max(h[i * 102:(i + 1) * 102], axis=0, keepdims=True) for i in range(4)]
    g = jnp.concatenate(gs, axis=0)
    g = jax.nn.relu(jnp.dot(g, l1[...], preferred_element_type=jnp.float32)
                    + v[9, 0:512][None, :])
    g = jax.nn.relu(jnp.dot(g, l2[...], preferred_element_type=jnp.float32)
                    + v[10, 0:256][None, :])
    g = jax.nn.relu(jnp.dot(g, l3[...], preferred_element_type=jnp.float32)
                    + v[11, 0:128][None, :])
    g = jnp.dot(g, l4[...], preferred_element_type=jnp.float32) + v[12, 0:2][None, :]
    mx = jnp.max(g, axis=1, keepdims=True)
    sh = g - mx
    out_ref[...] = sh - jnp.log(jnp.sum(jnp.exp(sh), axis=1, keepdims=True))


def _head_pallas(xcat, p):
    sa3 = p["sa3"]

    def padv(a, n=1024):
        return jnp.pad(a, (0, n - a.shape[0]))[None, :]

    vec = jnp.concatenate(
        [padv(sa3[i][k]) for i in range(3) for k in ("b", "g", "be")]
        + [padv(p["lin1"]["b"]), padv(p["lin2"]["b"]), padv(p["lin3"]["b"]),
           padv(p["lin4"]["b"])], axis=0)
    full = lambda a: pl.BlockSpec(a.shape, lambda: (0,) * a.ndim)
    args = (xcat, sa3[0]["W"], sa3[1]["W"], sa3[2]["W"],
            p["lin1"]["W"], p["lin2"]["W"], p["lin3"]["W"], p["lin4"]["W"], vec)
    return pl.pallas_call(
        _head_body,
        in_specs=[full(a) for a in args],
        out_specs=pl.BlockSpec((4, 2), lambda: (0, 0)),
        out_shape=jax.ShapeDtypeStruct((4, 2), jnp.float32),
    )(*args)


def kernel(data, params):
    p = params
    B, N = _B, _N
    coords = data[..., :3]
    knn_idx, knn_val = _topk_pallas(coords, coords, _K, flat_base=True)
    idxT = jnp.swapaxes(knn_idx, 1, 2).reshape(B, N * _K)
    valT = jnp.swapaxes(knn_val, 1, 2)
    ctable = jnp.pad(coords.reshape(B * N, 3), ((0, 0), (0, 13)))
    nc = _sc_gather(ctable, idxT.reshape(-1)).reshape(B, N * _K, 16)
    wenc = _enc_pack(p)
    x = _encoder_pallas(data, nc, valT, wenc)
    x = jnp.reshape(jnp.swapaxes(x, 1, 2), (B, N, 32))
    x, pos = _sa_module(x, coords, 0.2, 0.2, p["sa1"])
    x, pos = _sa_module(x, pos, 0.5, 0.2, p["sa1a"])
    x, pos = _sa_module(x, pos, 0.25, 0.4, p["sa2"])
    xcat = jnp.concatenate([x, pos], -1).reshape(B * 102, 259)
    return _head_pallas(xcat, p)
